# baseline (speedup 1.0000x reference)
import jax
import jax.numpy as jnp
from jax.experimental import pallas as pl
from jax.experimental.pallas import tpu as pltpu

_N = 1024
_D = 64
_H = 128
_BI = 1024


def _mp_kernel(xfull_ref, x_ref, adj_ref, w1_ref, b1_ref, w2_ref, b2_ref,
               wih_ref, whh_ref, bih_ref, bhh_ref, out_ref):
    xfull = xfull_ref[...]
    x = x_ref[...]
    adj = adj_ref[...]
    a_blk = (jnp.dot(x, w1_ref[0:_D, :], preferred_element_type=jnp.float32)
             + b1_ref[...])
    bfull = jnp.dot(xfull, w1_ref[_D:, :],
                    preferred_element_type=jnp.float32)
    adj_bf = adj.astype(jnp.bfloat16)
    rows = []
    for r in range(_BI):
        tb = (a_blk[r:r + 1, :] + bfull).astype(jnp.bfloat16)
        t = jnp.maximum(tb, jnp.bfloat16(0))
        rows.append(jnp.dot(adj_bf[r:r + 1, :], t,
                            preferred_element_type=jnp.float32))
    s = jnp.concatenate(rows, axis=0)
    deg = jnp.sum(adj, axis=1, keepdims=True)
    w2 = w2_ref[...].astype(jnp.bfloat16).astype(jnp.float32)
    msg = (jnp.dot(s, w2, preferred_element_type=jnp.float32,
                   precision=jax.lax.Precision.HIGHEST)
           + deg * b2_ref[...])
    gi = jnp.dot(msg, wih_ref[...], preferred_element_type=jnp.float32) \
        + bih_ref[...]
    gh = jnp.dot(x, whh_ref[...], preferred_element_type=jnp.float32) \
        + bhh_ref[...]
    r_g = jax.nn.sigmoid(gi[:, 0:_D] + gh[:, 0:_D])
    z_g = jax.nn.sigmoid(gi[:, _D:2 * _D] + gh[:, _D:2 * _D])
    n_g = jnp.tanh(gi[:, 2 * _D:] + r_g * gh[:, 2 * _D:])
    out_ref[...] = (1.0 - z_g) * n_g + z_g * x


def kernel(group_features, group_adjacency, W1, b1, W2, b2,
           W_ih, W_hh, b_ih, b_hh):
    x = group_features
    grid = (_N // _BI,)
    out = pl.pallas_call(
        _mp_kernel,
        grid=grid,
        in_specs=[
            pl.BlockSpec((_N, _D), lambda i: (0, 0)),
            pl.BlockSpec((_BI, _D), lambda i: (i, 0)),
            pl.BlockSpec((_BI, _N), lambda i: (i, 0)),
            pl.BlockSpec((2 * _D, _H), lambda i: (0, 0)),
            pl.BlockSpec((1, _H), lambda i: (0, 0)),
            pl.BlockSpec((_H, _D), lambda i: (0, 0)),
            pl.BlockSpec((1, _D), lambda i: (0, 0)),
            pl.BlockSpec((_D, 3 * _D), lambda i: (0, 0)),
            pl.BlockSpec((_D, 3 * _D), lambda i: (0, 0)),
            pl.BlockSpec((1, 3 * _D), lambda i: (0, 0)),
            pl.BlockSpec((1, 3 * _D), lambda i: (0, 0)),
        ],
        out_specs=pl.BlockSpec((_BI, _D), lambda i: (i, 0)),
        out_shape=jax.ShapeDtypeStruct((_N, _D), jnp.float32),
        compiler_params=pltpu.CompilerParams(
            dimension_semantics=("parallel",)),
    )(x, x, group_adjacency, W1, b1.reshape(1, _H), W2, b2.reshape(1, _D),
      W_ih.T, W_hh.T, b_ih.reshape(1, 3 * _D), b_hh.reshape(1, 3 * _D))
    return out


# speedup vs baseline: 1.0174x; 1.0174x over previous
import jax
import jax.numpy as jnp
from jax.experimental import pallas as pl
from jax.experimental.pallas import tpu as pltpu

_N = 1024
_D = 64
_H = 128
_BI = 512


def _mp_kernel(xfull_ref, x_ref, adj_ref, w1_ref, b1_ref, w2_ref, b2_ref,
               wih_ref, whh_ref, bih_ref, bhh_ref, out_ref):
    xfull = xfull_ref[...]
    x = x_ref[...]
    adj = adj_ref[...]
    a_blk = (jnp.dot(x, w1_ref[0:_D, :], preferred_element_type=jnp.float32)
             + b1_ref[...])
    bfull = jnp.dot(xfull, w1_ref[_D:, :],
                    preferred_element_type=jnp.float32)
    adj_bf = adj.astype(jnp.bfloat16)
    rows = []
    for r in range(_BI):
        tb = (a_blk[r:r + 1, :] + bfull).astype(jnp.bfloat16)
        t = jnp.maximum(tb, jnp.bfloat16(0))
        rows.append(jnp.dot(adj_bf[r:r + 1, :], t,
                            preferred_element_type=jnp.float32))
    s = jnp.concatenate(rows, axis=0)
    deg = jnp.sum(adj, axis=1, keepdims=True)
    w2 = w2_ref[...].astype(jnp.bfloat16).astype(jnp.float32)
    msg = (jnp.dot(s, w2, preferred_element_type=jnp.float32,
                   precision=jax.lax.Precision.HIGHEST)
           + deg * b2_ref[...])
    gi = jnp.dot(msg, wih_ref[...], preferred_element_type=jnp.float32) \
        + bih_ref[...]
    gh = jnp.dot(x, whh_ref[...], preferred_element_type=jnp.float32) \
        + bhh_ref[...]
    r_g = jax.nn.sigmoid(gi[:, 0:_D] + gh[:, 0:_D])
    z_g = jax.nn.sigmoid(gi[:, _D:2 * _D] + gh[:, _D:2 * _D])
    n_g = jnp.tanh(gi[:, 2 * _D:] + r_g * gh[:, 2 * _D:])
    out_ref[...] = (1.0 - z_g) * n_g + z_g * x


def kernel(group_features, group_adjacency, W1, b1, W2, b2,
           W_ih, W_hh, b_ih, b_hh):
    x = group_features
    grid = (_N // _BI,)
    out = pl.pallas_call(
        _mp_kernel,
        grid=grid,
        in_specs=[
            pl.BlockSpec((_N, _D), lambda i: (0, 0)),
            pl.BlockSpec((_BI, _D), lambda i: (i, 0)),
            pl.BlockSpec((_BI, _N), lambda i: (i, 0)),
            pl.BlockSpec((2 * _D, _H), lambda i: (0, 0)),
            pl.BlockSpec((1, _H), lambda i: (0, 0)),
            pl.BlockSpec((_H, _D), lambda i: (0, 0)),
            pl.BlockSpec((1, _D), lambda i: (0, 0)),
            pl.BlockSpec((_D, 3 * _D), lambda i: (0, 0)),
            pl.BlockSpec((_D, 3 * _D), lambda i: (0, 0)),
            pl.BlockSpec((1, 3 * _D), lambda i: (0, 0)),
            pl.BlockSpec((1, 3 * _D), lambda i: (0, 0)),
        ],
        out_specs=pl.BlockSpec((_BI, _D), lambda i: (i, 0)),
        out_shape=jax.ShapeDtypeStruct((_N, _D), jnp.float32),
        compiler_params=pltpu.CompilerParams(
            dimension_semantics=("parallel",)),
    )(x, x, group_adjacency, W1, b1.reshape(1, _H), W2, b2.reshape(1, _D),
      W_ih.T, W_hh.T, b_ih.reshape(1, 3 * _D), b_hh.reshape(1, 3 * _D))
    return out
